# baseline (speedup 1.0000x reference)
import functools

import jax
import jax.numpy as jnp
from jax import lax
from jax.experimental import pallas as pl
from jax.experimental.pallas import tpu as pltpu
from jax.experimental.pallas import tpu_sc as plsc

B = 4096
L_T = 50
L_D = 200
D_EMB = 64
D_BR = 16
HIDDEN = 128
OUT = 64

NC = 2
NS = 16
NW = NC * NS
IPW = B // NW

_MESH = dict(core_axis_name="c", subcore_axis_name="s")
_LINEAR = pltpu.CompilerParams(use_tc_tiling_on_sc=False)


def _row_sum2(rows_ref, n_rows):
    def body(r, accs):
        out = []
        for j in range(4):
            sl = pl.ds(j * 16, 16)
            out.append(accs[j] + (rows_ref[2 * r, sl] + rows_ref[2 * r + 1, sl]))
        return tuple(out)
    init = tuple(jnp.zeros((16,), jnp.float32) for _ in range(4))
    return lax.fori_loop(0, n_rows // 2, body, init)


def _sc_bags(t_idx, d_idx, t_tab, d_tab):
    @functools.partial(
        pl.kernel,
        compiler_params=_LINEAR,
        out_type=(
            jax.ShapeDtypeStruct((B, D_EMB), jnp.float32),
            jax.ShapeDtypeStruct((B, D_EMB), jnp.float32),
        ),
        mesh=plsc.VectorSubcoreMesh(**_MESH),
        scratch_types=[
            pltpu.VMEM((IPW, L_T), jnp.int32),
            pltpu.VMEM((IPW, L_D), jnp.int32),
            pltpu.VMEM((L_T, D_EMB), jnp.float32),
            pltpu.VMEM((L_T, D_EMB), jnp.float32),
            pltpu.VMEM((L_D, D_EMB), jnp.float32),
            pltpu.VMEM((L_D, D_EMB), jnp.float32),
            pltpu.VMEM((IPW, D_EMB), jnp.float32),
            pltpu.VMEM((IPW, D_EMB), jnp.float32),
            pltpu.SemaphoreType.DMA,
            pltpu.SemaphoreType.DMA,
            pltpu.SemaphoreType.DMA,
            pltpu.SemaphoreType.DMA,
        ],
    )
    def k(t_idx_hbm, d_idx_hbm, t_tab_hbm, d_tab_hbm,
          t_out, d_out,
          tiv, div, tbuf0, tbuf1, dbuf0, dbuf1, tacc, dacc,
          ts0, ts1, ds0, ds1):
        wid = lax.axis_index("s") * NC + lax.axis_index("c")
        base = wid * IPW

        pltpu.sync_copy(t_idx_hbm.at[pl.ds(base, IPW)], tiv)
        pltpu.sync_copy(d_idx_hbm.at[pl.ds(base, IPW)], div)

        def start(b, tbuf, dbuf, tsem, dsem):
            pltpu.async_copy(t_tab_hbm.at[tiv.at[b]], tbuf, tsem)
            pltpu.async_copy(d_tab_hbm.at[div.at[b, pl.ds(0, 128)]],
                             dbuf.at[pl.ds(0, 128)], dsem)
            pltpu.async_copy(d_tab_hbm.at[div.at[b, pl.ds(128, L_D - 128)]],
                             dbuf.at[pl.ds(128, L_D - 128)], dsem)

        def wait(tbuf, dbuf, tsem, dsem):
            pltpu.make_async_copy(t_tab_hbm.at[pl.ds(0, L_T)], tbuf, tsem).wait()
            pltpu.make_async_copy(d_tab_hbm.at[pl.ds(0, L_D)], dbuf, dsem).wait()

        def reduce(b, tbuf, dbuf):
            taccs = _row_sum2(tbuf, L_T)
            daccs = _row_sum2(dbuf, L_D)
            for j in range(4):
                tacc[b, pl.ds(j * 16, 16)] = taccs[j]
                dacc[b, pl.ds(j * 16, 16)] = daccs[j]

        start(0, tbuf0, dbuf0, ts0, ds0)

        @pl.loop(0, IPW, step=2)
        def _(b):
            start(b + 1, tbuf1, dbuf1, ts1, ds1)
            wait(tbuf0, dbuf0, ts0, ds0)
            reduce(b, tbuf0, dbuf0)

            @pl.when(b + 2 < IPW)
            def _():
                start(b + 2, tbuf0, dbuf0, ts0, ds0)

            wait(tbuf1, dbuf1, ts1, ds1)
            reduce(b + 1, tbuf1, dbuf1)

        pltpu.sync_copy(tacc, t_out.at[pl.ds(base, IPW)])
        pltpu.sync_copy(dacc, d_out.at[pl.ds(base, IPW)])

    return k(t_idx, d_idx, t_tab, d_tab)


def _sc_idbrand(id_idx, br_idx, id_tab2, br_tab8):
    @functools.partial(
        pl.kernel,
        compiler_params=pltpu.CompilerParams(use_tc_tiling_on_sc=True),
        out_type=(
            jax.ShapeDtypeStruct((B, 128), jnp.float32),
            jax.ShapeDtypeStruct((B, 128), jnp.float32),
        ),
        mesh=plsc.VectorSubcoreMesh(**_MESH),
        scratch_types=[
            pltpu.VMEM((IPW,), jnp.int32),
            pltpu.VMEM((IPW,), jnp.int32),
            pltpu.VMEM((IPW, 128), jnp.float32),
            pltpu.VMEM((IPW, 128), jnp.float32),
            pltpu.SemaphoreType.DMA,
            pltpu.SemaphoreType.DMA,
        ],
    )
    def k(id_idx_hbm, br_idx_hbm, id_tab_hbm, br_tab_hbm,
          id_out, br_out, idv, brv, idrows, brrows, sem1, sem2):
        wid = lax.axis_index("s") * NC + lax.axis_index("c")
        base = wid * IPW
        pltpu.sync_copy(id_idx_hbm.at[pl.ds(base, IPW)], idv)
        pltpu.sync_copy(br_idx_hbm.at[pl.ds(base, IPW)], brv)
        @pl.loop(0, IPW, step=16)
        def _(i):
            sl = pl.ds(i, 16)
            idv[sl] = jax.lax.shift_right_logical(idv[sl], 1)
            brv[sl] = jax.lax.shift_right_logical(brv[sl], 3)
        pltpu.async_copy(id_tab_hbm.at[idv], idrows, sem1)
        pltpu.async_copy(br_tab_hbm.at[brv], brrows, sem2)
        pltpu.make_async_copy(id_tab_hbm.at[pl.ds(0, IPW)], idrows, sem1).wait()
        pltpu.make_async_copy(br_tab_hbm.at[pl.ds(0, IPW)], brrows, sem2).wait()
        pltpu.sync_copy(idrows, id_out.at[pl.ds(base, IPW)])
        pltpu.sync_copy(brrows, br_out.at[pl.ds(base, IPW)])

    return k(id_idx, br_idx, id_tab2, br_tab8)


def _widen_body(tr, outr):
    y = jnp.transpose(tr[...])
    y3 = y.reshape(y.shape[0] // 2, 2, 64)
    outr[...] = jnp.concatenate([y3[:, 0, :], y3[:, 1, :]], axis=1)


def _tc_widen(table):
    v, d = table.shape
    assert d == 64
    t_t = table.T
    ch = 8192
    return pl.pallas_call(
        _widen_body,
        grid=(pl.cdiv(v, ch),),
        in_specs=[pl.BlockSpec((64, ch), lambda i: (0, i))],
        out_specs=pl.BlockSpec((ch // 2, 128), lambda i: (i, 0)),
        out_shape=jax.ShapeDtypeStruct((v // 2, 128), jnp.float32),
    )(t_t)


def _mlp_body(pidr, pbrr, idr, tr, dr, brr, w1r, b1r, w2r, b2r, outr):
    w1 = w1r[...]
    idp = idr[...]
    id_emb = jnp.where((pidr[...] & 1) == 0, idp[:, 0:D_EMB],
                       idp[:, D_EMB:2 * D_EMB])
    brp = brr[...]
    bmod = pbrr[...] & 7
    br_emb = jnp.zeros_like(brp[:, 0:D_BR])
    for kk in range(8):
        br_emb = jnp.where(bmod == kk, brp[:, kk * D_BR:(kk + 1) * D_BR], br_emb)
    h = jnp.dot(id_emb, w1[0:64], preferred_element_type=jnp.float32)
    h += (1.0 / L_T) * jnp.dot(tr[...], w1[64:128],
                               preferred_element_type=jnp.float32)
    h += (1.0 / L_D) * jnp.dot(dr[...], w1[128:192],
                               preferred_element_type=jnp.float32)
    h += jnp.dot(br_emb, w1[192:208], preferred_element_type=jnp.float32)
    h = jnp.maximum(h + b1r[...], 0.0)
    outr[...] = jnp.dot(h, w2r[...], preferred_element_type=jnp.float32) + b2r[...]


def _tc_mlp(pid, pbr, id_emb, t_sum, d_sum, br_emb, W1, b1, W2, b2):
    blk = 512
    return pl.pallas_call(
        _mlp_body,
        grid=(B // blk,),
        in_specs=[
            pl.BlockSpec((blk, 1), lambda i: (i, 0)),
            pl.BlockSpec((blk, 1), lambda i: (i, 0)),
            pl.BlockSpec((blk, 128), lambda i: (i, 0)),
            pl.BlockSpec((blk, D_EMB), lambda i: (i, 0)),
            pl.BlockSpec((blk, D_EMB), lambda i: (i, 0)),
            pl.BlockSpec((blk, 128), lambda i: (i, 0)),
            pl.BlockSpec((208, HIDDEN), lambda i: (0, 0)),
            pl.BlockSpec((1, HIDDEN), lambda i: (0, 0)),
            pl.BlockSpec((HIDDEN, OUT), lambda i: (0, 0)),
            pl.BlockSpec((1, OUT), lambda i: (0, 0)),
        ],
        out_specs=pl.BlockSpec((blk, OUT), lambda i: (i, 0)),
        out_shape=jax.ShapeDtypeStruct((B, OUT), jnp.float32),
    )(pid.reshape(B, 1), pbr.reshape(B, 1), id_emb, t_sum, d_sum, br_emb,
      W1, b1.reshape(1, HIDDEN), W2, b2.reshape(1, OUT))


def kernel(product_id, product_title, product_description, product_brand,
           id_table, title_table, desc_table, brand_table, W1, b1, W2, b2):
    t_sum, d_sum = _sc_bags(product_title.astype(jnp.int32),
                            product_description.astype(jnp.int32),
                            _tc_widen(title_table).reshape(-1, D_EMB),
                            _tc_widen(desc_table).reshape(-1, D_EMB))
    pid = product_id.astype(jnp.int32)
    pbr = product_brand.astype(jnp.int32)
    id_emb, br_emb = _sc_idbrand(pid, pbr,
                                 _tc_widen(id_table),
                                 brand_table.reshape(-1, 128))
    return _tc_mlp(pid, pbr, id_emb, t_sum, d_sum, br_emb, W1, b1, W2, b2)


# speedup vs baseline: 1.0398x; 1.0398x over previous
import functools

import jax
import jax.numpy as jnp
from jax import lax
from jax.experimental import pallas as pl
from jax.experimental.pallas import tpu as pltpu
from jax.experimental.pallas import tpu_sc as plsc

B = 4096
L_T = 50
L_D = 200
D_EMB = 64
D_BR = 16
HIDDEN = 128
OUT = 64

NC = 2
NS = 16
NW = NC * NS
IPW = B // NW

_MESH = dict(core_axis_name="c", subcore_axis_name="s")
_LINEAR = pltpu.CompilerParams(use_tc_tiling_on_sc=False)


def _row_sum2(rows_ref, n_rows):
    def body(r, accs):
        out = []
        for j in range(4):
            sl = pl.ds(j * 16, 16)
            out.append(accs[j] + (rows_ref[2 * r, sl] + rows_ref[2 * r + 1, sl]))
        return tuple(out)
    init = tuple(jnp.zeros((16,), jnp.float32) for _ in range(4))
    return lax.fori_loop(0, n_rows // 2, body, init)


def _sc_bags(t_idx, d_idx, t_tab, d_tab):
    @functools.partial(
        pl.kernel,
        compiler_params=_LINEAR,
        out_type=(
            jax.ShapeDtypeStruct((B, D_EMB), jnp.float32),
            jax.ShapeDtypeStruct((B, D_EMB), jnp.float32),
        ),
        mesh=plsc.VectorSubcoreMesh(**_MESH),
        scratch_types=[
            pltpu.VMEM((IPW, L_T), jnp.int32),
            pltpu.VMEM((IPW, L_D), jnp.int32),
            pltpu.VMEM((L_T, D_EMB), jnp.float32),
            pltpu.VMEM((L_T, D_EMB), jnp.float32),
            pltpu.VMEM((L_D, D_EMB), jnp.float32),
            pltpu.VMEM((L_D, D_EMB), jnp.float32),
            pltpu.VMEM((IPW, D_EMB), jnp.float32),
            pltpu.VMEM((IPW, D_EMB), jnp.float32),
            pltpu.SemaphoreType.DMA,
            pltpu.SemaphoreType.DMA,
            pltpu.SemaphoreType.DMA,
            pltpu.SemaphoreType.DMA,
        ],
    )
    def k(t_idx_hbm, d_idx_hbm, t_tab_hbm, d_tab_hbm,
          t_out, d_out,
          tiv, div, tbuf0, tbuf1, dbuf0, dbuf1, tacc, dacc,
          ts0, ts1, ds0, ds1):
        wid = lax.axis_index("s") * NC + lax.axis_index("c")
        base = wid * IPW

        pltpu.sync_copy(t_idx_hbm.at[pl.ds(base, IPW)], tiv)
        pltpu.sync_copy(d_idx_hbm.at[pl.ds(base, IPW)], div)

        def start(b, tbuf, dbuf, tsem, dsem):
            pltpu.async_copy(t_tab_hbm.at[tiv.at[b]], tbuf, tsem)
            pltpu.async_copy(d_tab_hbm.at[div.at[b, pl.ds(0, 128)]],
                             dbuf.at[pl.ds(0, 128)], dsem)
            pltpu.async_copy(d_tab_hbm.at[div.at[b, pl.ds(128, L_D - 128)]],
                             dbuf.at[pl.ds(128, L_D - 128)], dsem)

        def wait(tbuf, dbuf, tsem, dsem):
            pltpu.make_async_copy(t_tab_hbm.at[pl.ds(0, L_T)], tbuf, tsem).wait()
            pltpu.make_async_copy(d_tab_hbm.at[pl.ds(0, L_D)], dbuf, dsem).wait()

        def reduce(b, tbuf, dbuf):
            taccs = _row_sum2(tbuf, L_T)
            daccs = _row_sum2(dbuf, L_D)
            for j in range(4):
                tacc[b, pl.ds(j * 16, 16)] = taccs[j]
                dacc[b, pl.ds(j * 16, 16)] = daccs[j]

        start(0, tbuf0, dbuf0, ts0, ds0)

        @pl.loop(0, IPW, step=2)
        def _(b):
            start(b + 1, tbuf1, dbuf1, ts1, ds1)
            wait(tbuf0, dbuf0, ts0, ds0)
            reduce(b, tbuf0, dbuf0)

            @pl.when(b + 2 < IPW)
            def _():
                start(b + 2, tbuf0, dbuf0, ts0, ds0)

            wait(tbuf1, dbuf1, ts1, ds1)
            reduce(b + 1, tbuf1, dbuf1)

        pltpu.sync_copy(tacc, t_out.at[pl.ds(base, IPW)])
        pltpu.sync_copy(dacc, d_out.at[pl.ds(base, IPW)])

    return k(t_idx, d_idx, t_tab, d_tab)


def _sc_brand(br_idx, br_tab8):
    @functools.partial(
        pl.kernel,
        compiler_params=pltpu.CompilerParams(use_tc_tiling_on_sc=True),
        out_type=jax.ShapeDtypeStruct((B, 128), jnp.float32),
        mesh=plsc.VectorSubcoreMesh(**_MESH),
        scratch_types=[
            pltpu.VMEM((IPW,), jnp.int32),
            pltpu.VMEM((IPW, 128), jnp.float32),
            pltpu.SemaphoreType.DMA,
        ],
    )
    def k(br_idx_hbm, br_tab_hbm, br_out, brv, brrows, sem):
        wid = lax.axis_index("s") * NC + lax.axis_index("c")
        base = wid * IPW
        pltpu.sync_copy(br_idx_hbm.at[pl.ds(base, IPW)], brv)
        @pl.loop(0, IPW, step=16)
        def _(i):
            sl = pl.ds(i, 16)
            brv[sl] = jax.lax.shift_right_logical(brv[sl], 3)
        pltpu.async_copy(br_tab_hbm.at[brv], brrows, sem)
        pltpu.make_async_copy(br_tab_hbm.at[pl.ds(0, IPW)], brrows, sem).wait()
        pltpu.sync_copy(brrows, br_out.at[pl.ds(base, IPW)])

    return k(br_idx, br_tab8)


_XNB = 16


def _extract_body(pid_sm, t_ref, out_ref, buf, sem):
    i = pl.program_id(0)

    def issue(j):
        v = pid_sm[i * 128 + j]
        c = pl.multiple_of((v // 128) * 128, 128)
        return pltpu.make_async_copy(t_ref.at[:, pl.ds(c, 128)],
                                     buf.at[j % _XNB], sem)

    for j in range(_XNB):
        issue(j).start()
    lane = lax.broadcasted_iota(jnp.int32, (D_EMB, 128), 1)
    for j0 in range(0, 128, 4):
        cols = []
        for j in range(j0, j0 + 4):
            issue(j).wait()
            l = pid_sm[i * 128 + j] % 128
            sel = jnp.where(lane == l, buf[j % _XNB], 0.0)
            cols.append(jnp.sum(sel, axis=1, keepdims=True))
        for k, j in enumerate(range(j0, j0 + 4)):
            out_ref[:, j:j + 1] = cols[k]
        for j in range(j0, j0 + 4):
            if j + _XNB < 128:
                issue(j + _XNB).start()


def _tc_id_extract(id_table, pid):
    t_t = id_table.T
    return pl.pallas_call(
        _extract_body,
        grid_spec=pltpu.PrefetchScalarGridSpec(
            num_scalar_prefetch=1,
            grid=(B // 128,),
            in_specs=[pl.BlockSpec(memory_space=pl.ANY)],
            out_specs=pl.BlockSpec((D_EMB, 128), lambda i, pid_ref: (0, i)),
            scratch_shapes=[pltpu.VMEM((_XNB, D_EMB, 128), jnp.float32),
                            pltpu.SemaphoreType.DMA],
        ),
        out_shape=jax.ShapeDtypeStruct((D_EMB, B), jnp.float32),
    )(pid, t_t)


def _widen_body(tr, outr):
    y = jnp.transpose(tr[...])
    y3 = y.reshape(y.shape[0] // 2, 2, 64)
    outr[...] = jnp.concatenate([y3[:, 0, :], y3[:, 1, :]], axis=1)


def _tc_widen(table):
    v, d = table.shape
    assert d == 64
    t_t = table.T
    ch = 8192
    return pl.pallas_call(
        _widen_body,
        grid=(pl.cdiv(v, ch),),
        in_specs=[pl.BlockSpec((64, ch), lambda i: (0, i))],
        out_specs=pl.BlockSpec((ch // 2, 128), lambda i: (i, 0)),
        out_shape=jax.ShapeDtypeStruct((v // 2, 128), jnp.float32),
    )(t_t)


def _mlp_body(pbrr, idcr, tr, dr, brr, w1r, b1r, w2r, b2r, outr):
    w1 = w1r[...]
    h = lax.dot_general(idcr[...], w1[0:64], (((0,), (0,)), ((), ())),
                        preferred_element_type=jnp.float32)
    brp = brr[...]
    bmod = pbrr[...] & 7
    br_emb = jnp.zeros_like(brp[:, 0:D_BR])
    for kk in range(8):
        br_emb = jnp.where(bmod == kk, brp[:, kk * D_BR:(kk + 1) * D_BR], br_emb)
    h += (1.0 / L_T) * jnp.dot(tr[...], w1[64:128],
                               preferred_element_type=jnp.float32)
    h += (1.0 / L_D) * jnp.dot(dr[...], w1[128:192],
                               preferred_element_type=jnp.float32)
    h += jnp.dot(br_emb, w1[192:208], preferred_element_type=jnp.float32)
    h = jnp.maximum(h + b1r[...], 0.0)
    outr[...] = jnp.dot(h, w2r[...], preferred_element_type=jnp.float32) + b2r[...]


def _tc_mlp(pbr, id_cols, t_sum, d_sum, br_emb, W1, b1, W2, b2):
    blk = 512
    return pl.pallas_call(
        _mlp_body,
        grid=(B // blk,),
        in_specs=[
            pl.BlockSpec((blk, 1), lambda i: (i, 0)),
            pl.BlockSpec((D_EMB, blk), lambda i: (0, i)),
            pl.BlockSpec((blk, D_EMB), lambda i: (i, 0)),
            pl.BlockSpec((blk, D_EMB), lambda i: (i, 0)),
            pl.BlockSpec((blk, 128), lambda i: (i, 0)),
            pl.BlockSpec((208, HIDDEN), lambda i: (0, 0)),
            pl.BlockSpec((1, HIDDEN), lambda i: (0, 0)),
            pl.BlockSpec((HIDDEN, OUT), lambda i: (0, 0)),
            pl.BlockSpec((1, OUT), lambda i: (0, 0)),
        ],
        out_specs=pl.BlockSpec((blk, OUT), lambda i: (i, 0)),
        out_shape=jax.ShapeDtypeStruct((B, OUT), jnp.float32),
    )(pbr.reshape(B, 1), id_cols, t_sum, d_sum, br_emb,
      W1, b1.reshape(1, HIDDEN), W2, b2.reshape(1, OUT))


def kernel(product_id, product_title, product_description, product_brand,
           id_table, title_table, desc_table, brand_table, W1, b1, W2, b2):
    t_sum, d_sum = _sc_bags(product_title.astype(jnp.int32),
                            product_description.astype(jnp.int32),
                            _tc_widen(title_table).reshape(-1, D_EMB),
                            _tc_widen(desc_table).reshape(-1, D_EMB))
    pid = product_id.astype(jnp.int32)
    pbr = product_brand.astype(jnp.int32)
    id_cols = _tc_id_extract(id_table, pid)
    br_emb = _sc_brand(pbr, brand_table.reshape(-1, 128))
    return _tc_mlp(pbr, id_cols, t_sum, d_sum, br_emb, W1, b1, W2, b2)


# speedup vs baseline: 1.8298x; 1.7598x over previous
import functools

import jax
import jax.numpy as jnp
from jax import lax
from jax.experimental import pallas as pl
from jax.experimental.pallas import tpu as pltpu
from jax.experimental.pallas import tpu_sc as plsc

B = 4096
L_T = 50
L_D = 200
D_EMB = 64
D_BR = 16
HIDDEN = 128
OUT = 64

NC = 2
NS = 16
NW = NC * NS
IPW = B // NW

_MESH = dict(core_axis_name="c", subcore_axis_name="s")
_LINEAR = pltpu.CompilerParams(use_tc_tiling_on_sc=False)


def _row_sum2(rows_ref, n_rows):
    def body(r, accs):
        out = []
        for j in range(4):
            sl = pl.ds(j * 16, 16)
            out.append(accs[j] + (rows_ref[2 * r, sl] + rows_ref[2 * r + 1, sl]))
        return tuple(out)
    init = tuple(jnp.zeros((16,), jnp.float32) for _ in range(4))
    return lax.fori_loop(0, n_rows // 2, body, init)


def _sc_bags(t_idx, d_idx, t_tab, d_tab):
    @functools.partial(
        pl.kernel,
        compiler_params=_LINEAR,
        out_type=(
            jax.ShapeDtypeStruct((B, D_EMB), jnp.float32),
            jax.ShapeDtypeStruct((B, D_EMB), jnp.float32),
        ),
        mesh=plsc.VectorSubcoreMesh(**_MESH),
        scratch_types=[
            pltpu.VMEM((IPW, L_T), jnp.int32),
            pltpu.VMEM((IPW, L_D), jnp.int32),
            pltpu.VMEM((L_T, D_EMB), jnp.float32),
            pltpu.VMEM((L_T, D_EMB), jnp.float32),
            pltpu.VMEM((L_D, D_EMB), jnp.float32),
            pltpu.VMEM((L_D, D_EMB), jnp.float32),
            pltpu.VMEM((IPW, D_EMB), jnp.float32),
            pltpu.VMEM((IPW, D_EMB), jnp.float32),
            pltpu.SemaphoreType.DMA,
            pltpu.SemaphoreType.DMA,
            pltpu.SemaphoreType.DMA,
            pltpu.SemaphoreType.DMA,
        ],
    )
    def k(t_idx_hbm, d_idx_hbm, t_tab_hbm, d_tab_hbm,
          t_out, d_out,
          tiv, div, tbuf0, tbuf1, dbuf0, dbuf1, tacc, dacc,
          ts0, ts1, ds0, ds1):
        wid = lax.axis_index("s") * NC + lax.axis_index("c")
        base = wid * IPW

        pltpu.sync_copy(t_idx_hbm.at[pl.ds(base, IPW)], tiv)
        pltpu.sync_copy(d_idx_hbm.at[pl.ds(base, IPW)], div)

        def start(b, tbuf, dbuf, tsem, dsem):
            pltpu.async_copy(t_tab_hbm.at[tiv.at[b]], tbuf, tsem)
            pltpu.async_copy(d_tab_hbm.at[div.at[b, pl.ds(0, 128)]],
                             dbuf.at[pl.ds(0, 128)], dsem)
            pltpu.async_copy(d_tab_hbm.at[div.at[b, pl.ds(128, L_D - 128)]],
                             dbuf.at[pl.ds(128, L_D - 128)], dsem)

        def wait(tbuf, dbuf, tsem, dsem):
            pltpu.make_async_copy(t_tab_hbm.at[pl.ds(0, L_T)], tbuf, tsem).wait()
            pltpu.make_async_copy(d_tab_hbm.at[pl.ds(0, L_D)], dbuf, dsem).wait()

        def reduce(b, tbuf, dbuf):
            taccs = _row_sum2(tbuf, L_T)
            daccs = _row_sum2(dbuf, L_D)
            for j in range(4):
                tacc[b, pl.ds(j * 16, 16)] = taccs[j]
                dacc[b, pl.ds(j * 16, 16)] = daccs[j]

        start(0, tbuf0, dbuf0, ts0, ds0)

        @pl.loop(0, IPW, step=2)
        def _(b):
            start(b + 1, tbuf1, dbuf1, ts1, ds1)
            wait(tbuf0, dbuf0, ts0, ds0)
            reduce(b, tbuf0, dbuf0)

            @pl.when(b + 2 < IPW)
            def _():
                start(b + 2, tbuf0, dbuf0, ts0, ds0)

            wait(tbuf1, dbuf1, ts1, ds1)
            reduce(b + 1, tbuf1, dbuf1)

        pltpu.sync_copy(tacc, t_out.at[pl.ds(base, IPW)])
        pltpu.sync_copy(dacc, d_out.at[pl.ds(base, IPW)])

    return k(t_idx, d_idx, t_tab, d_tab)


def _sc_brand(br_idx, br_tab8):
    @functools.partial(
        pl.kernel,
        compiler_params=pltpu.CompilerParams(use_tc_tiling_on_sc=True),
        out_type=jax.ShapeDtypeStruct((B, 128), jnp.float32),
        mesh=plsc.VectorSubcoreMesh(**_MESH),
        scratch_types=[
            pltpu.VMEM((IPW,), jnp.int32),
            pltpu.VMEM((IPW, 128), jnp.float32),
            pltpu.SemaphoreType.DMA,
        ],
    )
    def k(br_idx_hbm, br_tab_hbm, br_out, brv, brrows, sem):
        wid = lax.axis_index("s") * NC + lax.axis_index("c")
        base = wid * IPW
        pltpu.sync_copy(br_idx_hbm.at[pl.ds(base, IPW)], brv)
        @pl.loop(0, IPW, step=16)
        def _(i):
            sl = pl.ds(i, 16)
            brv[sl] = jax.lax.shift_right_logical(brv[sl], 3)
        pltpu.async_copy(br_tab_hbm.at[brv], brrows, sem)
        pltpu.make_async_copy(br_tab_hbm.at[pl.ds(0, IPW)], brrows, sem).wait()
        pltpu.sync_copy(brrows, br_out.at[pl.ds(base, IPW)])

    return k(br_idx, br_tab8)


def _extract_body(pid_sm, t_ref, out_ref, buf, sem):
    i = pl.program_id(0)

    def issue(j):
        v = pid_sm[i * 128 + j]
        c = pl.multiple_of((v // 128) * 128, 128)
        return pltpu.make_async_copy(t_ref.at[:, pl.ds(c, 128)],
                                     buf.at[j], sem)

    for j in range(128):
        issue(j).start()
    for j in range(128):
        issue(j).wait()
    lane = lax.broadcasted_iota(jnp.int32, (D_EMB, 128), 1)
    for j0 in range(0, 128, 4):
        cols = []
        for j in range(j0, j0 + 4):
            l = pid_sm[i * 128 + j] % 128
            sel = jnp.where(lane == l, buf[j], 0.0)
            cols.append(jnp.sum(sel, axis=1, keepdims=True))
        for k, j in enumerate(range(j0, j0 + 4)):
            out_ref[:, j:j + 1] = cols[k]


def _tc_id_extract(id_table, pid):
    t_t = id_table.T
    return pl.pallas_call(
        _extract_body,
        grid_spec=pltpu.PrefetchScalarGridSpec(
            num_scalar_prefetch=1,
            grid=(B // 128,),
            in_specs=[pl.BlockSpec(memory_space=pl.ANY)],
            out_specs=pl.BlockSpec((D_EMB, 128), lambda i, pid_ref: (0, i)),
            scratch_shapes=[pltpu.VMEM((128, D_EMB, 128), jnp.float32),
                            pltpu.SemaphoreType.DMA],
        ),
        out_shape=jax.ShapeDtypeStruct((D_EMB, B), jnp.float32),
    )(pid, t_t)


def _widen_body(tr, outr):
    y = jnp.transpose(tr[...])
    y3 = y.reshape(y.shape[0] // 2, 2, 64)
    outr[...] = jnp.concatenate([y3[:, 0, :], y3[:, 1, :]], axis=1)


def _tc_widen(table):
    v, d = table.shape
    assert d == 64
    t_t = table.T
    ch = 8192
    return pl.pallas_call(
        _widen_body,
        grid=(pl.cdiv(v, ch),),
        in_specs=[pl.BlockSpec((64, ch), lambda i: (0, i))],
        out_specs=pl.BlockSpec((ch // 2, 128), lambda i: (i, 0)),
        out_shape=jax.ShapeDtypeStruct((v // 2, 128), jnp.float32),
    )(t_t)


def _mlp_body(pbrr, idcr, tr, dr, brr, w1r, b1r, w2r, b2r, outr):
    w1 = w1r[...]
    h = lax.dot_general(idcr[...], w1[0:64], (((0,), (0,)), ((), ())),
                        preferred_element_type=jnp.float32)
    brp = brr[...]
    bmod = pbrr[...] & 7
    br_emb = jnp.zeros_like(brp[:, 0:D_BR])
    for kk in range(8):
        br_emb = jnp.where(bmod == kk, brp[:, kk * D_BR:(kk + 1) * D_BR], br_emb)
    h += (1.0 / L_T) * jnp.dot(tr[...], w1[64:128],
                               preferred_element_type=jnp.float32)
    h += (1.0 / L_D) * jnp.dot(dr[...], w1[128:192],
                               preferred_element_type=jnp.float32)
    h += jnp.dot(br_emb, w1[192:208], preferred_element_type=jnp.float32)
    h = jnp.maximum(h + b1r[...], 0.0)
    outr[...] = jnp.dot(h, w2r[...], preferred_element_type=jnp.float32) + b2r[...]


def _tc_mlp(pbr, id_cols, t_sum, d_sum, br_emb, W1, b1, W2, b2):
    blk = 512
    return pl.pallas_call(
        _mlp_body,
        grid=(B // blk,),
        in_specs=[
            pl.BlockSpec((blk, 1), lambda i: (i, 0)),
            pl.BlockSpec((D_EMB, blk), lambda i: (0, i)),
            pl.BlockSpec((blk, D_EMB), lambda i: (i, 0)),
            pl.BlockSpec((blk, D_EMB), lambda i: (i, 0)),
            pl.BlockSpec((blk, 128), lambda i: (i, 0)),
            pl.BlockSpec((208, HIDDEN), lambda i: (0, 0)),
            pl.BlockSpec((1, HIDDEN), lambda i: (0, 0)),
            pl.BlockSpec((HIDDEN, OUT), lambda i: (0, 0)),
            pl.BlockSpec((1, OUT), lambda i: (0, 0)),
        ],
        out_specs=pl.BlockSpec((blk, OUT), lambda i: (i, 0)),
        out_shape=jax.ShapeDtypeStruct((B, OUT), jnp.float32),
    )(pbr.reshape(B, 1), id_cols, t_sum, d_sum, br_emb,
      W1, b1.reshape(1, HIDDEN), W2, b2.reshape(1, OUT))


def kernel(product_id, product_title, product_description, product_brand,
           id_table, title_table, desc_table, brand_table, W1, b1, W2, b2):
    t_sum, d_sum = _sc_bags(product_title.astype(jnp.int32),
                            product_description.astype(jnp.int32),
                            _tc_widen(title_table).reshape(-1, D_EMB),
                            _tc_widen(desc_table).reshape(-1, D_EMB))
    pid = product_id.astype(jnp.int32)
    pbr = product_brand.astype(jnp.int32)
    id_cols = _tc_id_extract(id_table, pid)
    br_emb = _sc_brand(pbr, brand_table.reshape(-1, 128))
    return _tc_mlp(pbr, id_cols, t_sum, d_sum, br_emb, W1, b1, W2, b2)


# speedup vs baseline: 1.8356x; 1.0032x over previous
import functools

import jax
import jax.numpy as jnp
from jax import lax
from jax.experimental import pallas as pl
from jax.experimental.pallas import tpu as pltpu
from jax.experimental.pallas import tpu_sc as plsc

B = 4096
L_T = 50
L_D = 200
D_EMB = 64
D_BR = 16
HIDDEN = 128
OUT = 64

NC = 2
NS = 16
NW = NC * NS
IPW = B // NW

_MESH = dict(core_axis_name="c", subcore_axis_name="s")
_LINEAR = pltpu.CompilerParams(use_tc_tiling_on_sc=False)


def _row_sum2(rows_ref, n_rows):
    def body(r, accs):
        out = []
        for j in range(4):
            sl = pl.ds(j * 16, 16)
            out.append(accs[j] + (rows_ref[2 * r, sl] + rows_ref[2 * r + 1, sl]))
        return tuple(out)
    init = tuple(jnp.zeros((16,), jnp.float32) for _ in range(4))
    return lax.fori_loop(0, n_rows // 2, body, init)


def _sc_bag(idx, tab, n_rows):
    chunks = [(0, min(128, n_rows))]
    if n_rows > 128:
        chunks.append((128, n_rows - 128))

    @functools.partial(
        pl.kernel,
        compiler_params=_LINEAR,
        out_type=jax.ShapeDtypeStruct((B, D_EMB), jnp.float32),
        mesh=plsc.VectorSubcoreMesh(**_MESH),
        scratch_types=[
            pltpu.VMEM((IPW, n_rows), jnp.int32),
            pltpu.VMEM((n_rows, D_EMB), jnp.float32),
            pltpu.VMEM((n_rows, D_EMB), jnp.float32),
            pltpu.VMEM((IPW, D_EMB), jnp.float32),
            pltpu.SemaphoreType.DMA,
            pltpu.SemaphoreType.DMA,
        ],
    )
    def k(idx_hbm, tab_hbm, out, iv, buf0, buf1, acc, s0, s1):
        wid = lax.axis_index("s") * NC + lax.axis_index("c")
        base = wid * IPW

        pltpu.sync_copy(idx_hbm.at[pl.ds(base, IPW)], iv)

        def start(b, buf, sem):
            for off, ln in chunks:
                pltpu.async_copy(tab_hbm.at[iv.at[b, pl.ds(off, ln)]],
                                 buf.at[pl.ds(off, ln)], sem)

        def wait(buf, sem):
            pltpu.make_async_copy(tab_hbm.at[pl.ds(0, n_rows)], buf, sem).wait()

        def reduce(b, buf):
            accs = _row_sum2(buf, n_rows)
            for j in range(4):
                acc[b, pl.ds(j * 16, 16)] = accs[j]

        start(0, buf0, s0)

        @pl.loop(0, IPW, step=2)
        def _(b):
            start(b + 1, buf1, s1)
            wait(buf0, s0)
            reduce(b, buf0)

            @pl.when(b + 2 < IPW)
            def _():
                start(b + 2, buf0, s0)

            wait(buf1, s1)
            reduce(b + 1, buf1)

        pltpu.sync_copy(acc, out.at[pl.ds(base, IPW)])

    return k(idx, tab)


def _sc_brand(br_idx, br_tab8):
    @functools.partial(
        pl.kernel,
        compiler_params=pltpu.CompilerParams(use_tc_tiling_on_sc=True),
        out_type=jax.ShapeDtypeStruct((B, 128), jnp.float32),
        mesh=plsc.VectorSubcoreMesh(**_MESH),
        scratch_types=[
            pltpu.VMEM((IPW,), jnp.int32),
            pltpu.VMEM((IPW, 128), jnp.float32),
            pltpu.SemaphoreType.DMA,
        ],
    )
    def k(br_idx_hbm, br_tab_hbm, br_out, brv, brrows, sem):
        wid = lax.axis_index("s") * NC + lax.axis_index("c")
        base = wid * IPW
        pltpu.sync_copy(br_idx_hbm.at[pl.ds(base, IPW)], brv)
        @pl.loop(0, IPW, step=16)
        def _(i):
            sl = pl.ds(i, 16)
            brv[sl] = jax.lax.shift_right_logical(brv[sl], 3)
        pltpu.async_copy(br_tab_hbm.at[brv], brrows, sem)
        pltpu.make_async_copy(br_tab_hbm.at[pl.ds(0, IPW)], brrows, sem).wait()
        pltpu.sync_copy(brrows, br_out.at[pl.ds(base, IPW)])

    return k(br_idx, br_tab8)


def _extract_body(pid_sm, t_ref, out_ref, buf, sem):
    i = pl.program_id(0)

    def issue(j):
        v = pid_sm[i * 128 + j]
        c = pl.multiple_of((v // 128) * 128, 128)
        return pltpu.make_async_copy(t_ref.at[:, pl.ds(c, 128)],
                                     buf.at[j], sem)

    for j in range(128):
        issue(j).start()
    for j in range(128):
        issue(j).wait()
    lane = lax.broadcasted_iota(jnp.int32, (D_EMB, 128), 1)
    for j0 in range(0, 128, 4):
        cols = []
        for j in range(j0, j0 + 4):
            l = pid_sm[i * 128 + j] % 128
            sel = jnp.where(lane == l, buf[j], 0.0)
            cols.append(jnp.sum(sel, axis=1, keepdims=True))
        for k, j in enumerate(range(j0, j0 + 4)):
            out_ref[:, j:j + 1] = cols[k]


def _tc_id_extract(id_table, pid):
    t_t = id_table.T
    return pl.pallas_call(
        _extract_body,
        grid_spec=pltpu.PrefetchScalarGridSpec(
            num_scalar_prefetch=1,
            grid=(B // 128,),
            in_specs=[pl.BlockSpec(memory_space=pl.ANY)],
            out_specs=pl.BlockSpec((D_EMB, 128), lambda i, pid_ref: (0, i)),
            scratch_shapes=[pltpu.VMEM((128, D_EMB, 128), jnp.float32),
                            pltpu.SemaphoreType.DMA],
        ),
        out_shape=jax.ShapeDtypeStruct((D_EMB, B), jnp.float32),
    )(pid, t_t)


def _widen_body(tr, outr):
    y = jnp.transpose(tr[...])
    y3 = y.reshape(y.shape[0] // 2, 2, 64)
    outr[...] = jnp.concatenate([y3[:, 0, :], y3[:, 1, :]], axis=1)


def _tc_widen(table):
    v, d = table.shape
    assert d == 64
    t_t = table.T
    ch = 8192
    return pl.pallas_call(
        _widen_body,
        grid=(pl.cdiv(v, ch),),
        in_specs=[pl.BlockSpec((64, ch), lambda i: (0, i))],
        out_specs=pl.BlockSpec((ch // 2, 128), lambda i: (i, 0)),
        out_shape=jax.ShapeDtypeStruct((v // 2, 128), jnp.float32),
    )(t_t)


def _mlp_body(pbrr, idcr, tr, dr, brr, w1r, b1r, w2r, b2r, outr):
    w1 = w1r[...]
    h = lax.dot_general(idcr[...], w1[0:64], (((0,), (0,)), ((), ())),
                        preferred_element_type=jnp.float32)
    brp = brr[...]
    bmod = pbrr[...] & 7
    br_emb = jnp.zeros_like(brp[:, 0:D_BR])
    for kk in range(8):
        br_emb = jnp.where(bmod == kk, brp[:, kk * D_BR:(kk + 1) * D_BR], br_emb)
    h += (1.0 / L_T) * jnp.dot(tr[...], w1[64:128],
                               preferred_element_type=jnp.float32)
    h += (1.0 / L_D) * jnp.dot(dr[...], w1[128:192],
                               preferred_element_type=jnp.float32)
    h += jnp.dot(br_emb, w1[192:208], preferred_element_type=jnp.float32)
    h = jnp.maximum(h + b1r[...], 0.0)
    outr[...] = jnp.dot(h, w2r[...], preferred_element_type=jnp.float32) + b2r[...]


def _tc_mlp(pbr, id_cols, t_sum, d_sum, br_emb, W1, b1, W2, b2):
    blk = 512
    return pl.pallas_call(
        _mlp_body,
        grid=(B // blk,),
        in_specs=[
            pl.BlockSpec((blk, 1), lambda i: (i, 0)),
            pl.BlockSpec((D_EMB, blk), lambda i: (0, i)),
            pl.BlockSpec((blk, D_EMB), lambda i: (i, 0)),
            pl.BlockSpec((blk, D_EMB), lambda i: (i, 0)),
            pl.BlockSpec((blk, 128), lambda i: (i, 0)),
            pl.BlockSpec((208, HIDDEN), lambda i: (0, 0)),
            pl.BlockSpec((1, HIDDEN), lambda i: (0, 0)),
            pl.BlockSpec((HIDDEN, OUT), lambda i: (0, 0)),
            pl.BlockSpec((1, OUT), lambda i: (0, 0)),
        ],
        out_specs=pl.BlockSpec((blk, OUT), lambda i: (i, 0)),
        out_shape=jax.ShapeDtypeStruct((B, OUT), jnp.float32),
    )(pbr.reshape(B, 1), id_cols, t_sum, d_sum, br_emb,
      W1, b1.reshape(1, HIDDEN), W2, b2.reshape(1, OUT))


def kernel(product_id, product_title, product_description, product_brand,
           id_table, title_table, desc_table, brand_table, W1, b1, W2, b2):
    d_sum = _sc_bag(product_description.astype(jnp.int32),
                    _tc_widen(desc_table).reshape(-1, D_EMB), L_D)
    t_sum = _sc_bag(product_title.astype(jnp.int32),
                    _tc_widen(title_table).reshape(-1, D_EMB), L_T)
    pid = product_id.astype(jnp.int32)
    pbr = product_brand.astype(jnp.int32)
    id_cols = _tc_id_extract(id_table, pid)
    br_emb = _sc_brand(pbr, brand_table.reshape(-1, 128))
    return _tc_mlp(pbr, id_cols, t_sum, d_sum, br_emb, W1, b1, W2, b2)
